# Initial kernel scaffold; baseline (speedup 1.0000x reference)
import jax
import jax.numpy as jnp
from jax.experimental import pallas as pl


def kernel(x1, x2):
    raise NotImplementedError("write your pallas kernel here")


# speedup vs baseline: 1044.4195x; 1044.4195x over previous
import functools

import jax
import jax.numpy as jnp
from jax.experimental import pallas as pl


def _dist_tile_kernel(x1_ref, x2_ref, out_ref):
    a = x1_ref[...]
    b = x2_ref[...]
    g = jax.lax.dot_general(
        a, b, (((1,), (1,)), ((), ())), preferred_element_type=jnp.float32
    )
    na = jnp.sum(a * a, axis=1, keepdims=True)
    nb = jnp.sum(b * b, axis=1, keepdims=True).T
    d2 = na + nb - 2.0 * g
    out_ref[...] = jnp.sqrt(jnp.maximum(d2, 0.0) + 1e-12)


@functools.partial(jax.jit, static_argnames=("bm",))
def _pairwise_dist(x1, x2, bm=128):
    n1, d = x1.shape
    n2 = x2.shape[0]
    grid = (n1 // bm,)
    return pl.pallas_call(
        _dist_tile_kernel,
        grid=grid,
        in_specs=[
            pl.BlockSpec((bm, d), lambda i: (i, 0)),
            pl.BlockSpec((n2, d), lambda i: (0, 0)),
        ],
        out_specs=pl.BlockSpec((bm, n2), lambda i: (i, 0)),
        out_shape=jax.ShapeDtypeStruct((n1, n2), jnp.float32),
    )(x1, x2)


def kernel(x1, x2):
    return _pairwise_dist(x1, x2)


# speedup vs baseline: 1226.5040x; 1.1743x over previous
import functools

import jax
import jax.numpy as jnp
from jax.experimental import pallas as pl


def _dist_tile_kernel(x1_ref, x2_ref, out_ref):
    a = x1_ref[...]
    b = x2_ref[...]
    g = jax.lax.dot_general(
        a, b, (((1,), (1,)), ((), ())), preferred_element_type=jnp.float32
    )
    na = jnp.sum(a * a, axis=1, keepdims=True)
    nb = jnp.sum(b * b, axis=1, keepdims=True).T
    d2 = na + nb - 2.0 * g
    out_ref[...] = jnp.sqrt(jnp.maximum(d2, 0.0) + 1e-12)


@functools.partial(jax.jit, static_argnames=("bm",))
def _pairwise_dist(x1, x2, bm=256):
    n1, d = x1.shape
    n2 = x2.shape[0]
    grid = (n1 // bm,)
    return pl.pallas_call(
        _dist_tile_kernel,
        grid=grid,
        in_specs=[
            pl.BlockSpec((bm, d), lambda i: (i, 0)),
            pl.BlockSpec((n2, d), lambda i: (0, 0)),
        ],
        out_specs=pl.BlockSpec((bm, n2), lambda i: (i, 0)),
        out_shape=jax.ShapeDtypeStruct((n1, n2), jnp.float32),
    )(x1, x2)


def kernel(x1, x2):
    return _pairwise_dist(x1, x2)


# speedup vs baseline: 1242.5477x; 1.0131x over previous
import functools

import jax
import jax.numpy as jnp
from jax.experimental import pallas as pl


def _dist_tile_kernel(x1_ref, x2_ref, out_ref):
    a = x1_ref[...]
    b = x2_ref[...]
    g = jax.lax.dot_general(
        a, b, (((1,), (1,)), ((), ())), preferred_element_type=jnp.float32
    )
    na = jnp.sum(a * a, axis=1, keepdims=True)
    nb = jnp.sum(b * b, axis=1, keepdims=True).T
    d2 = na + nb - 2.0 * g
    out_ref[...] = jnp.sqrt(jnp.maximum(d2, 0.0) + 1e-12)


@functools.partial(jax.jit, static_argnames=("bm",))
def _pairwise_dist(x1, x2, bm=512):
    n1, d = x1.shape
    n2 = x2.shape[0]
    grid = (n1 // bm,)
    return pl.pallas_call(
        _dist_tile_kernel,
        grid=grid,
        in_specs=[
            pl.BlockSpec((bm, d), lambda i: (i, 0)),
            pl.BlockSpec((n2, d), lambda i: (0, 0)),
        ],
        out_specs=pl.BlockSpec((bm, n2), lambda i: (i, 0)),
        out_shape=jax.ShapeDtypeStruct((n1, n2), jnp.float32),
    )(x1, x2)


def kernel(x1, x2):
    return _pairwise_dist(x1, x2)
